# Initial kernel scaffold; baseline (speedup 1.0000x reference)
import jax
import jax.numpy as jnp
from jax.experimental import pallas as pl


def kernel(x, edge_index, W1, b1, W2, b2):
    raise NotImplementedError("write your pallas kernel here")


# speedup vs baseline: 17.0556x; 17.0556x over previous
import functools

import jax
import jax.numpy as jnp
from jax import lax
from jax.experimental import pallas as pl
from jax.experimental.pallas import tpu as pltpu
from jax.experimental.pallas import tpu_sc as plsc

N_NODES = 10000
N_EDGES = 320000
D_IN = 128
D_HID = 16
D_OUT = 64

NC = 2
NS = 16
NW = NC * NS
CHUNK = 128
CPW = -(-N_EDGES // (CHUNK * NW))
N_CHUNKS = CPW * NW
E_PAD = N_CHUNKS * CHUNK
N_PAD = 10240
RPS = N_PAD // NS

_HIGH = jax.lax.Precision.HIGHEST
_MESH = dict(core_axis_name="c", subcore_axis_name="s")
_DOT = (((1,), (0,)), ((), ()))
_SC_PARAMS = pltpu.CompilerParams(use_tc_tiling_on_sc=False)


def _sc_degree(dst2d, ones, zeros):
    @functools.partial(
        pl.kernel,
        out_type=jax.ShapeDtypeStruct((NC, N_PAD, D_HID), jnp.float32),
        mesh=plsc.VectorSubcoreMesh(**_MESH),
        scratch_types=[
            pltpu.VMEM((CHUNK,), jnp.int32),
            pltpu.VMEM((CHUNK, D_HID), jnp.float32),
            pltpu.VMEM_SHARED((N_PAD, D_HID), jnp.float32),
        ],
        compiler_params=_SC_PARAMS,
    )
    def deg_kernel(dst_hbm, ones_hbm, zeros_hbm, out_hbm, idx_v, ones_v, acc_sh):
        c = lax.axis_index("c")
        s = lax.axis_index("s")
        w = c * NS + s
        row0 = s * RPS
        pltpu.sync_copy(zeros_hbm.at[pl.ds(row0, RPS)],
                        acc_sh.at[pl.ds(row0, RPS)])
        pltpu.sync_copy(ones_hbm, ones_v)
        plsc.subcore_barrier()

        @pl.loop(0, CPW)
        def _(i):
            pltpu.sync_copy(dst_hbm.at[w * CPW + i], idx_v)
            pltpu.sync_copy(ones_v, acc_sh.at[idx_v], add=True)

        plsc.subcore_barrier()
        pltpu.sync_copy(acc_sh.at[pl.ds(row0, RPS)],
                        out_hbm.at[c, pl.ds(row0, RPS)])

    return deg_kernel(dst2d, ones, zeros)


def _sc_scatter(table, src2d, dst2d, zeros, d):
    @functools.partial(
        pl.kernel,
        out_type=jax.ShapeDtypeStruct((NC, N_PAD, d), jnp.float32),
        mesh=plsc.VectorSubcoreMesh(**_MESH),
        scratch_types=[
            pltpu.VMEM((CHUNK,), jnp.int32),
            pltpu.VMEM((CHUNK,), jnp.int32),
            pltpu.VMEM((CHUNK, d), jnp.float32),
            pltpu.VMEM_SHARED((N_PAD, d), jnp.float32),
            pltpu.SemaphoreType.DMA,
        ],
        compiler_params=_SC_PARAMS,
    )
    def scat_kernel(tab_hbm, src_hbm, dst_hbm, zeros_hbm, out_hbm,
                    si_v, di_v, rows_v, acc_sh, sem):
        c = lax.axis_index("c")
        s = lax.axis_index("s")
        w = c * NS + s
        row0 = s * RPS
        pltpu.sync_copy(zeros_hbm.at[pl.ds(row0, RPS)],
                        acc_sh.at[pl.ds(row0, RPS)])
        plsc.subcore_barrier()

        @pl.loop(0, CPW)
        def _(i):
            cid = w * CPW + i
            pltpu.sync_copy(src_hbm.at[cid], si_v)
            pltpu.sync_copy(dst_hbm.at[cid], di_v)
            pltpu.async_copy(tab_hbm.at[si_v], rows_v, sem).wait()
            pltpu.sync_copy(rows_v, acc_sh.at[di_v], add=True)

        plsc.subcore_barrier()
        pltpu.sync_copy(acc_sh.at[pl.ds(row0, RPS)],
                        out_hbm.at[c, pl.ds(row0, RPS)])

    return scat_kernel(table, src2d, dst2d, zeros)


_BR = 1000


def _tc_mm(x, w):
    m, k = x.shape
    n = w.shape[1]

    def body(x_ref, w_ref, o_ref):
        o_ref[...] = lax.dot_general(x_ref[...], w_ref[...], _DOT,
                                     precision=_HIGH,
                                     preferred_element_type=jnp.float32)

    return pl.pallas_call(
        body,
        grid=(m // _BR,),
        in_specs=[pl.BlockSpec((_BR, k), lambda i: (i, 0)),
                  pl.BlockSpec((k, n), lambda i: (0, 0))],
        out_specs=pl.BlockSpec((_BR, n), lambda i: (i, 0)),
        out_shape=jax.ShapeDtypeStruct((m, n), jnp.float32),
    )(x, w)


def _tc_norm(deg_p, h1):
    def body(dp_ref, h_ref, u_ref, hn_ref):
        u = lax.rsqrt(dp_ref[0] + dp_ref[1] + 1.0)
        u_ref[...] = u
        hn_ref[...] = u * h_ref[...]

    return pl.pallas_call(
        body,
        grid=(N_NODES // _BR,),
        in_specs=[pl.BlockSpec((NC, _BR, D_HID), lambda i: (0, i, 0)),
                  pl.BlockSpec((_BR, D_HID), lambda i: (i, 0))],
        out_specs=[pl.BlockSpec((_BR, D_HID), lambda i: (i, 0)),
                   pl.BlockSpec((_BR, D_HID), lambda i: (i, 0))],
        out_shape=[jax.ShapeDtypeStruct((N_NODES, D_HID), jnp.float32),
                   jax.ShapeDtypeStruct((N_NODES, D_HID), jnp.float32)],
    )(deg_p, h1)


def _tc_mid(p1, hn1, u16, b1, w2):
    def body(p_ref, hn_ref, u_ref, b_ref, w_ref, hn2_ref, u64_ref):
        pre = u_ref[...] * (p_ref[0] + p_ref[1] + hn_ref[...]) + b_ref[...]
        o1 = jnp.maximum(pre, 0.0)
        h2 = lax.dot_general(o1, w_ref[...], _DOT, precision=_HIGH,
                             preferred_element_type=jnp.float32)
        sel = (lax.broadcasted_iota(jnp.int32, (D_HID, D_OUT), 0) == 0)
        u64 = lax.dot_general(u_ref[...], sel.astype(jnp.float32), _DOT,
                              precision=_HIGH, preferred_element_type=jnp.float32)
        u64_ref[...] = u64
        hn2_ref[...] = u64 * h2

    return pl.pallas_call(
        body,
        grid=(N_NODES // _BR,),
        in_specs=[pl.BlockSpec((NC, _BR, D_HID), lambda i: (0, i, 0)),
                  pl.BlockSpec((_BR, D_HID), lambda i: (i, 0)),
                  pl.BlockSpec((_BR, D_HID), lambda i: (i, 0)),
                  pl.BlockSpec((1, D_HID), lambda i: (0, 0)),
                  pl.BlockSpec((D_HID, D_OUT), lambda i: (0, 0))],
        out_specs=[pl.BlockSpec((_BR, D_OUT), lambda i: (i, 0)),
                   pl.BlockSpec((_BR, D_OUT), lambda i: (i, 0))],
        out_shape=[jax.ShapeDtypeStruct((N_NODES, D_OUT), jnp.float32),
                   jax.ShapeDtypeStruct((N_NODES, D_OUT), jnp.float32)],
    )(p1, hn1, u16, b1, w2)


def _tc_final(p2, hn2, u64, b2):
    def body(p_ref, hn_ref, u_ref, b_ref, o_ref):
        o_ref[...] = u_ref[...] * (p_ref[0] + p_ref[1] + hn_ref[...]) + b_ref[...]

    return pl.pallas_call(
        body,
        grid=(N_NODES // _BR,),
        in_specs=[pl.BlockSpec((NC, _BR, D_OUT), lambda i: (0, i, 0)),
                  pl.BlockSpec((_BR, D_OUT), lambda i: (i, 0)),
                  pl.BlockSpec((_BR, D_OUT), lambda i: (i, 0)),
                  pl.BlockSpec((1, D_OUT), lambda i: (0, 0))],
        out_specs=pl.BlockSpec((_BR, D_OUT), lambda i: (i, 0)),
        out_shape=jax.ShapeDtypeStruct((N_NODES, D_OUT), jnp.float32),
    )(p2, hn2, u64, b2)


def kernel(x, edge_index, W1, b1, W2, b2):
    src = edge_index[0].astype(jnp.int32)
    dst = edge_index[1].astype(jnp.int32)
    pad = E_PAD - N_EDGES
    src2d = jnp.concatenate([src, jnp.zeros((pad,), jnp.int32)]).reshape(N_CHUNKS, CHUNK)
    dst2d = jnp.concatenate([dst, jnp.full((pad,), N_NODES, jnp.int32)]).reshape(N_CHUNKS, CHUNK)
    zeros16 = jnp.zeros((N_PAD, D_HID), jnp.float32)
    zeros64 = jnp.zeros((N_PAD, D_OUT), jnp.float32)
    ones = jnp.ones((CHUNK, D_HID), jnp.float32)

    deg_p = _sc_degree(dst2d, ones, zeros16)
    h1 = _tc_mm(x, W1)
    u16, hn1 = _tc_norm(deg_p, h1)
    p1 = _sc_scatter(hn1, src2d, dst2d, zeros16, D_HID)
    hn2, u64 = _tc_mid(p1, hn1, u16, b1.reshape(1, D_HID), W2)
    p2 = _sc_scatter(hn2, src2d, dst2d, zeros64, D_OUT)
    return _tc_final(p2, hn2, u64, b2.reshape(1, D_OUT))


# speedup vs baseline: 29.6544x; 1.7387x over previous
import functools

import jax
import jax.numpy as jnp
from jax import lax
from jax.experimental import pallas as pl
from jax.experimental.pallas import tpu as pltpu
from jax.experimental.pallas import tpu_sc as plsc

N_NODES = 10000
N_EDGES = 320000
D_IN = 128
D_HID = 16
D_OUT = 64

NC = 2
NS = 16
NW = NC * NS
CHUNK = 128
CPW = -(-N_EDGES // (CHUNK * NW))
N_CHUNKS = CPW * NW
E_PAD = N_CHUNKS * CHUNK
N_PAD = 10240
RPS = N_PAD // NS

_HIGH = jax.lax.Precision.HIGHEST
_MESH = dict(core_axis_name="c", subcore_axis_name="s")
_DOT = (((1,), (0,)), ((), ()))
_SC_PARAMS = pltpu.CompilerParams(use_tc_tiling_on_sc=False)


def _sc_degree(dst2d, ones, zeros):
    @functools.partial(
        pl.kernel,
        out_type=jax.ShapeDtypeStruct((NC, N_PAD, D_HID), jnp.float32),
        mesh=plsc.VectorSubcoreMesh(**_MESH),
        scratch_types=[
            pltpu.VMEM((CPW, CHUNK), jnp.int32),
            pltpu.VMEM((CHUNK, D_HID), jnp.float32),
            pltpu.VMEM_SHARED((N_PAD, D_HID), jnp.float32),
        ],
        compiler_params=_SC_PARAMS,
    )
    def deg_kernel(dst_hbm, ones_hbm, zeros_hbm, out_hbm, di_v, ones_v, acc_sh):
        c = lax.axis_index("c")
        s = lax.axis_index("s")
        w = c * NS + s
        row0 = s * RPS
        pltpu.sync_copy(dst_hbm.at[pl.ds(w * CPW, CPW)], di_v)
        pltpu.sync_copy(zeros_hbm.at[pl.ds(row0, RPS)],
                        acc_sh.at[pl.ds(row0, RPS)])
        pltpu.sync_copy(ones_hbm, ones_v)
        plsc.subcore_barrier()

        @pl.loop(0, CPW)
        def _(i):
            pltpu.sync_copy(ones_v, acc_sh.at[di_v.at[i]], add=True)

        plsc.subcore_barrier()
        pltpu.sync_copy(acc_sh.at[pl.ds(row0, RPS)],
                        out_hbm.at[c, pl.ds(row0, RPS)])

    return deg_kernel(dst2d, ones, zeros)


def _sc_scatter(table, src2d, dst2d, zeros, d):
    @functools.partial(
        pl.kernel,
        out_type=jax.ShapeDtypeStruct((NC, N_PAD, d), jnp.float32),
        mesh=plsc.VectorSubcoreMesh(**_MESH),
        scratch_types=[
            pltpu.VMEM((CPW, CHUNK), jnp.int32),
            pltpu.VMEM((CPW, CHUNK), jnp.int32),
            pltpu.VMEM((CHUNK, d), jnp.float32),
            pltpu.VMEM((CHUNK, d), jnp.float32),
            pltpu.VMEM_SHARED((N_PAD, d), jnp.float32),
            pltpu.SemaphoreType.DMA,
            pltpu.SemaphoreType.DMA,
        ],
        compiler_params=_SC_PARAMS,
    )
    def scat_kernel(tab_hbm, src_hbm, dst_hbm, zeros_hbm, out_hbm,
                    si_v, di_v, buf_a, buf_b, acc_sh, sem_a, sem_b):
        c = lax.axis_index("c")
        s = lax.axis_index("s")
        w = c * NS + s
        row0 = s * RPS
        pltpu.sync_copy(src_hbm.at[pl.ds(w * CPW, CPW)], si_v)
        pltpu.sync_copy(dst_hbm.at[pl.ds(w * CPW, CPW)], di_v)
        pltpu.sync_copy(zeros_hbm.at[pl.ds(row0, RPS)],
                        acc_sh.at[pl.ds(row0, RPS)])
        plsc.subcore_barrier()

        def gather(i, buf, sem):
            pltpu.async_copy(tab_hbm.at[si_v.at[i]], buf, sem)

        def drain_scatter(i, buf, sem):
            pltpu.make_async_copy(tab_hbm.at[si_v.at[i]], buf, sem).wait()
            pltpu.sync_copy(buf, acc_sh.at[di_v.at[i]], add=True)

        gather(0, buf_a, sem_a)

        @pl.loop(0, CPW - 1, step=2)
        def _(i):
            gather(i + 1, buf_b, sem_b)
            drain_scatter(i, buf_a, sem_a)
            gather(i + 2, buf_a, sem_a)
            drain_scatter(i + 1, buf_b, sem_b)

        drain_scatter(CPW - 1, buf_a, sem_a)

        plsc.subcore_barrier()
        pltpu.sync_copy(acc_sh.at[pl.ds(row0, RPS)],
                        out_hbm.at[c, pl.ds(row0, RPS)])

    return scat_kernel(table, src2d, dst2d, zeros)


_BR = 1000


def _tc_mm(x, w):
    m, k = x.shape
    n = w.shape[1]

    def body(x_ref, w_ref, o_ref):
        o_ref[...] = lax.dot_general(x_ref[...], w_ref[...], _DOT,
                                     precision=_HIGH,
                                     preferred_element_type=jnp.float32)

    return pl.pallas_call(
        body,
        grid=(m // _BR,),
        in_specs=[pl.BlockSpec((_BR, k), lambda i: (i, 0)),
                  pl.BlockSpec((k, n), lambda i: (0, 0))],
        out_specs=pl.BlockSpec((_BR, n), lambda i: (i, 0)),
        out_shape=jax.ShapeDtypeStruct((m, n), jnp.float32),
    )(x, w)


def _tc_norm(deg_p, h1):
    def body(dp_ref, h_ref, u_ref, hn_ref):
        u = lax.rsqrt(dp_ref[0] + dp_ref[1] + 1.0)
        u_ref[...] = u
        hn_ref[...] = u * h_ref[...]

    return pl.pallas_call(
        body,
        grid=(N_NODES // _BR,),
        in_specs=[pl.BlockSpec((NC, _BR, D_HID), lambda i: (0, i, 0)),
                  pl.BlockSpec((_BR, D_HID), lambda i: (i, 0))],
        out_specs=[pl.BlockSpec((_BR, D_HID), lambda i: (i, 0)),
                   pl.BlockSpec((_BR, D_HID), lambda i: (i, 0))],
        out_shape=[jax.ShapeDtypeStruct((N_NODES, D_HID), jnp.float32),
                   jax.ShapeDtypeStruct((N_NODES, D_HID), jnp.float32)],
    )(deg_p, h1)


def _tc_mid(p1, hn1, u16, b1, w2):
    def body(p_ref, hn_ref, u_ref, b_ref, w_ref, hn2_ref, u64_ref):
        pre = u_ref[...] * (p_ref[0] + p_ref[1] + hn_ref[...]) + b_ref[...]
        o1 = jnp.maximum(pre, 0.0)
        h2 = lax.dot_general(o1, w_ref[...], _DOT, precision=_HIGH,
                             preferred_element_type=jnp.float32)
        sel = (lax.broadcasted_iota(jnp.int32, (D_HID, D_OUT), 0) == 0)
        u64 = lax.dot_general(u_ref[...], sel.astype(jnp.float32), _DOT,
                              precision=_HIGH, preferred_element_type=jnp.float32)
        u64_ref[...] = u64
        hn2_ref[...] = u64 * h2

    return pl.pallas_call(
        body,
        grid=(N_NODES // _BR,),
        in_specs=[pl.BlockSpec((NC, _BR, D_HID), lambda i: (0, i, 0)),
                  pl.BlockSpec((_BR, D_HID), lambda i: (i, 0)),
                  pl.BlockSpec((_BR, D_HID), lambda i: (i, 0)),
                  pl.BlockSpec((1, D_HID), lambda i: (0, 0)),
                  pl.BlockSpec((D_HID, D_OUT), lambda i: (0, 0))],
        out_specs=[pl.BlockSpec((_BR, D_OUT), lambda i: (i, 0)),
                   pl.BlockSpec((_BR, D_OUT), lambda i: (i, 0))],
        out_shape=[jax.ShapeDtypeStruct((N_NODES, D_OUT), jnp.float32),
                   jax.ShapeDtypeStruct((N_NODES, D_OUT), jnp.float32)],
    )(p1, hn1, u16, b1, w2)


def _tc_final(p2, hn2, u64, b2):
    def body(p_ref, hn_ref, u_ref, b_ref, o_ref):
        o_ref[...] = u_ref[...] * (p_ref[0] + p_ref[1] + hn_ref[...]) + b_ref[...]

    return pl.pallas_call(
        body,
        grid=(N_NODES // _BR,),
        in_specs=[pl.BlockSpec((NC, _BR, D_OUT), lambda i: (0, i, 0)),
                  pl.BlockSpec((_BR, D_OUT), lambda i: (i, 0)),
                  pl.BlockSpec((_BR, D_OUT), lambda i: (i, 0)),
                  pl.BlockSpec((1, D_OUT), lambda i: (0, 0))],
        out_specs=pl.BlockSpec((_BR, D_OUT), lambda i: (i, 0)),
        out_shape=jax.ShapeDtypeStruct((N_NODES, D_OUT), jnp.float32),
    )(p2, hn2, u64, b2)


def kernel(x, edge_index, W1, b1, W2, b2):
    src = edge_index[0].astype(jnp.int32)
    dst = edge_index[1].astype(jnp.int32)
    pad = E_PAD - N_EDGES
    src2d = jnp.concatenate([src, jnp.zeros((pad,), jnp.int32)]).reshape(N_CHUNKS, CHUNK)
    dst2d = jnp.concatenate([dst, jnp.full((pad,), N_NODES, jnp.int32)]).reshape(N_CHUNKS, CHUNK)
    zeros16 = jnp.zeros((N_PAD, D_HID), jnp.float32)
    zeros64 = jnp.zeros((N_PAD, D_OUT), jnp.float32)
    ones = jnp.ones((CHUNK, D_HID), jnp.float32)

    deg_p = _sc_degree(dst2d, ones, zeros16)
    h1 = _tc_mm(x, W1)
    u16, hn1 = _tc_norm(deg_p, h1)
    p1 = _sc_scatter(hn1, src2d, dst2d, zeros16, D_HID)
    hn2, u64 = _tc_mid(p1, hn1, u16, b1.reshape(1, D_HID), W2)
    p2 = _sc_scatter(hn2, src2d, dst2d, zeros64, D_OUT)
    return _tc_final(p2, hn2, u64, b2.reshape(1, D_OUT))


# speedup vs baseline: 36.3530x; 1.2259x over previous
import functools

import jax
import jax.numpy as jnp
from jax import lax
from jax.experimental import pallas as pl
from jax.experimental.pallas import tpu as pltpu
from jax.experimental.pallas import tpu_sc as plsc

N_NODES = 10000
N_EDGES = 320000
D_IN = 128
D_HID = 16
D_OUT = 64

NC = 2
NS = 16
NW = NC * NS
CHUNK = 128
CPW = -(-N_EDGES // (CHUNK * NW))
N_CHUNKS = CPW * NW
E_PAD = N_CHUNKS * CHUNK
N_PAD = 10240
RPS = N_PAD // NS

_HIGH = jax.lax.Precision.HIGHEST
_MESH = dict(core_axis_name="c", subcore_axis_name="s")
_DOT = (((1,), (0,)), ((), ()))
_SC_PARAMS = pltpu.CompilerParams(use_tc_tiling_on_sc=False)


def _sc_degree(dst2d, ones, zeros):
    @functools.partial(
        pl.kernel,
        out_type=jax.ShapeDtypeStruct((NC, N_PAD, D_HID), jnp.float32),
        mesh=plsc.VectorSubcoreMesh(**_MESH),
        scratch_types=[
            pltpu.VMEM((CPW, CHUNK), jnp.int32),
            pltpu.VMEM((CHUNK, D_HID), jnp.float32),
            pltpu.VMEM_SHARED((N_PAD, D_HID), jnp.float32),
        ],
        compiler_params=_SC_PARAMS,
    )
    def deg_kernel(dst_hbm, ones_hbm, zeros_hbm, out_hbm, di_v, ones_v, acc_sh):
        c = lax.axis_index("c")
        s = lax.axis_index("s")
        w = c * NS + s
        row0 = s * RPS
        pltpu.sync_copy(dst_hbm.at[pl.ds(w * CPW, CPW)], di_v)
        pltpu.sync_copy(zeros_hbm.at[pl.ds(row0, RPS)],
                        acc_sh.at[pl.ds(row0, RPS)])
        pltpu.sync_copy(ones_hbm, ones_v)
        plsc.subcore_barrier()

        @pl.loop(0, CPW)
        def _(i):
            pltpu.sync_copy(ones_v, acc_sh.at[di_v.at[i]], add=True)

        plsc.subcore_barrier()
        pltpu.sync_copy(acc_sh.at[pl.ds(row0, RPS)],
                        out_hbm.at[c, pl.ds(row0, RPS)])

    return deg_kernel(dst2d, ones, zeros)


def _sc_scatter(table, src2d, dst2d, zeros, d):
    @functools.partial(
        pl.kernel,
        out_type=jax.ShapeDtypeStruct((NC, N_PAD, d), jnp.bfloat16),
        mesh=plsc.VectorSubcoreMesh(**_MESH),
        scratch_types=[
            pltpu.VMEM((CPW, CHUNK), jnp.int32),
            pltpu.VMEM((CPW, CHUNK), jnp.int32),
            pltpu.VMEM((CHUNK, d), jnp.bfloat16),
            pltpu.VMEM((CHUNK, d), jnp.bfloat16),
            pltpu.VMEM_SHARED((N_PAD, d), jnp.bfloat16),
            pltpu.SemaphoreType.DMA,
            pltpu.SemaphoreType.DMA,
        ],
        compiler_params=_SC_PARAMS,
    )
    def scat_kernel(tab_hbm, src_hbm, dst_hbm, zeros_hbm, out_hbm,
                    si_v, di_v, buf_a, buf_b, acc_sh, sem_a, sem_b):
        c = lax.axis_index("c")
        s = lax.axis_index("s")
        w = c * NS + s
        row0 = s * RPS
        pltpu.sync_copy(src_hbm.at[pl.ds(w * CPW, CPW)], si_v)
        pltpu.sync_copy(dst_hbm.at[pl.ds(w * CPW, CPW)], di_v)
        pltpu.sync_copy(zeros_hbm.at[pl.ds(row0, RPS)],
                        acc_sh.at[pl.ds(row0, RPS)])
        plsc.subcore_barrier()

        def gather(i, buf, sem):
            pltpu.async_copy(tab_hbm.at[si_v.at[i]], buf, sem)

        def drain_scatter(i, buf, sem):
            pltpu.make_async_copy(tab_hbm.at[si_v.at[i]], buf, sem).wait()
            pltpu.sync_copy(buf, acc_sh.at[di_v.at[i]], add=True)

        gather(0, buf_a, sem_a)

        @pl.loop(0, CPW - 1, step=2)
        def _(i):
            gather(i + 1, buf_b, sem_b)
            drain_scatter(i, buf_a, sem_a)
            gather(i + 2, buf_a, sem_a)
            drain_scatter(i + 1, buf_b, sem_b)

        drain_scatter(CPW - 1, buf_a, sem_a)

        plsc.subcore_barrier()
        pltpu.sync_copy(acc_sh.at[pl.ds(row0, RPS)],
                        out_hbm.at[c, pl.ds(row0, RPS)])

    return scat_kernel(table, src2d, dst2d, zeros)


_BR = 1000


def _tc_mm(x, w):
    m, k = x.shape
    n = w.shape[1]

    def body(x_ref, w_ref, o_ref):
        o_ref[...] = lax.dot_general(x_ref[...], w_ref[...], _DOT,
                                     precision=_HIGH,
                                     preferred_element_type=jnp.float32)

    return pl.pallas_call(
        body,
        grid=(m // _BR,),
        in_specs=[pl.BlockSpec((_BR, k), lambda i: (i, 0)),
                  pl.BlockSpec((k, n), lambda i: (0, 0))],
        out_specs=pl.BlockSpec((_BR, n), lambda i: (i, 0)),
        out_shape=jax.ShapeDtypeStruct((m, n), jnp.float32),
    )(x, w)


def _tc_norm(deg_p, h1):
    def body(dp_ref, h_ref, u_ref, hn_ref):
        u = lax.rsqrt(dp_ref[0] + dp_ref[1] + 1.0)
        u_ref[...] = u
        hn_ref[...] = (u * h_ref[...]).astype(jnp.bfloat16)

    return pl.pallas_call(
        body,
        grid=(N_NODES // _BR,),
        in_specs=[pl.BlockSpec((NC, _BR, D_HID), lambda i: (0, i, 0)),
                  pl.BlockSpec((_BR, D_HID), lambda i: (i, 0))],
        out_specs=[pl.BlockSpec((_BR, D_HID), lambda i: (i, 0)),
                   pl.BlockSpec((_BR, D_HID), lambda i: (i, 0))],
        out_shape=[jax.ShapeDtypeStruct((N_NODES, D_HID), jnp.float32),
                   jax.ShapeDtypeStruct((N_NODES, D_HID), jnp.bfloat16)],
    )(deg_p, h1)


def _tc_mid(p1, hn1, u16, b1, w2):
    def body(p_ref, hn_ref, u_ref, b_ref, w_ref, hn2_ref, u64_ref):
        s1 = (p_ref[0] + p_ref[1]).astype(jnp.float32) + hn_ref[...].astype(jnp.float32)
        pre = u_ref[...] * s1 + b_ref[...]
        o1 = jnp.maximum(pre, 0.0)
        h2 = lax.dot_general(o1, w_ref[...], _DOT, precision=_HIGH,
                             preferred_element_type=jnp.float32)
        sel = (lax.broadcasted_iota(jnp.int32, (D_HID, D_OUT), 0) == 0)
        u64 = lax.dot_general(u_ref[...], sel.astype(jnp.float32), _DOT,
                              precision=_HIGH, preferred_element_type=jnp.float32)
        u64_ref[...] = u64
        hn2_ref[...] = (u64 * h2).astype(jnp.bfloat16)

    return pl.pallas_call(
        body,
        grid=(N_NODES // _BR,),
        in_specs=[pl.BlockSpec((NC, _BR, D_HID), lambda i: (0, i, 0)),
                  pl.BlockSpec((_BR, D_HID), lambda i: (i, 0)),
                  pl.BlockSpec((_BR, D_HID), lambda i: (i, 0)),
                  pl.BlockSpec((1, D_HID), lambda i: (0, 0)),
                  pl.BlockSpec((D_HID, D_OUT), lambda i: (0, 0))],
        out_specs=[pl.BlockSpec((_BR, D_OUT), lambda i: (i, 0)),
                   pl.BlockSpec((_BR, D_OUT), lambda i: (i, 0))],
        out_shape=[jax.ShapeDtypeStruct((N_NODES, D_OUT), jnp.bfloat16),
                   jax.ShapeDtypeStruct((N_NODES, D_OUT), jnp.float32)],
    )(p1, hn1, u16, b1, w2)


def _tc_final(p2, hn2, u64, b2):
    def body(p_ref, hn_ref, u_ref, b_ref, o_ref):
        s2 = (p_ref[0] + p_ref[1]).astype(jnp.float32) + hn_ref[...].astype(jnp.float32)
        o_ref[...] = u_ref[...] * s2 + b_ref[...]

    return pl.pallas_call(
        body,
        grid=(N_NODES // _BR,),
        in_specs=[pl.BlockSpec((NC, _BR, D_OUT), lambda i: (0, i, 0)),
                  pl.BlockSpec((_BR, D_OUT), lambda i: (i, 0)),
                  pl.BlockSpec((_BR, D_OUT), lambda i: (i, 0)),
                  pl.BlockSpec((1, D_OUT), lambda i: (0, 0))],
        out_specs=pl.BlockSpec((_BR, D_OUT), lambda i: (i, 0)),
        out_shape=jax.ShapeDtypeStruct((N_NODES, D_OUT), jnp.float32),
    )(p2, hn2, u64, b2)


def kernel(x, edge_index, W1, b1, W2, b2):
    src = edge_index[0].astype(jnp.int32)
    dst = edge_index[1].astype(jnp.int32)
    pad = E_PAD - N_EDGES
    src2d = jnp.concatenate([src, jnp.zeros((pad,), jnp.int32)]).reshape(N_CHUNKS, CHUNK)
    dst2d = jnp.concatenate([dst, jnp.full((pad,), N_NODES, jnp.int32)]).reshape(N_CHUNKS, CHUNK)
    zerosdeg = jnp.zeros((N_PAD, D_HID), jnp.float32)
    zeros16 = jnp.zeros((N_PAD, D_HID), jnp.bfloat16)
    zeros64 = jnp.zeros((N_PAD, D_OUT), jnp.bfloat16)
    ones = jnp.ones((CHUNK, D_HID), jnp.float32)

    deg_p = _sc_degree(dst2d, ones, zerosdeg)
    h1 = _tc_mm(x, W1)
    u16, hn1 = _tc_norm(deg_p, h1)
    p1 = _sc_scatter(hn1, src2d, dst2d, zeros16, D_HID)
    hn2, u64 = _tc_mid(p1, hn1, u16, b1.reshape(1, D_HID), W2)
    p2 = _sc_scatter(hn2, src2d, dst2d, zeros64, D_OUT)
    return _tc_final(p2, hn2, u64, b2.reshape(1, D_OUT))
